# Initial kernel scaffold; baseline (speedup 1.0000x reference)
import jax
import jax.numpy as jnp
from jax.experimental import pallas as pl


def kernel(x, edge_attr, edge_index, batch, lin_W, lin_b, bm1_W, bm1_b, bm2_W, bm2_b, conv_b, gru_Wih, gru_Whh, gru_bih, gru_bhh, lstm_Wih, lstm_Whh, lstm_bih, lstm_bhh, bn_g, bn_b, fc1_W, fc1_b, bn1_g, bn1_b, fc2_W, fc2_b, bn2_g, bn2_b, pred_W, pred_b):
    raise NotImplementedError("write your pallas kernel here")


# speedup vs baseline: 1.1155x; 1.1155x over previous
import functools
import jax
import jax.numpy as jnp
from jax import lax
from jax.experimental import pallas as pl
from jax.experimental.pallas import tpu as pltpu
from jax.experimental.pallas import tpu_sc as plsc

N = 10000
E = 160000
B = 256
ND = 128
ED = 16
AD = 32
CD = 128
OUT = 34
NUM_EMBEDS = 3
STEPS = 3

CHUNK = 128
NCHUNK = E // CHUNK
ET = 640
SUBS = 16
ROWS_PER_SUB = N // SUBS

_f32 = jnp.float32
_bf16 = jnp.bfloat16
_HI = lax.Precision.HIGHEST


def _d(a, b):
    return lax.dot_general(a.astype(_bf16), b.astype(_bf16),
                           (((a.ndim - 1,), (0,)), ((), ())),
                           preferred_element_type=_f32)


def _lin_body(x_ref, w_ref, b_ref, o_ref):
    o_ref[...] = jnp.maximum(_d(x_ref[...], w_ref[...]) + b_ref[...], 0.0)


def _lin_relu(x, w, b):
    return pl.pallas_call(
        _lin_body,
        out_shape=jax.ShapeDtypeStruct((x.shape[0], w.shape[1]), _f32),
    )(x, w, b)


def _sc_gather_body(nw, x_hbm, src_hbm, out_hbm, idx_v, rows_v, sem):
    c = lax.axis_index("c")
    s = lax.axis_index("s")
    nc = nw // SUBS
    wid = s * nc + c

    def body(j, carry):
        cc = j * nw + wid

        @pl.when(cc < NCHUNK)
        def _():
            pltpu.sync_copy(src_hbm.at[cc], idx_v)
            pltpu.async_copy(x_hbm.at[idx_v], rows_v, sem).wait()
            pltpu.sync_copy(rows_v, out_hbm.at[pl.ds(cc * CHUNK, CHUNK)])

        return carry

    lax.fori_loop(0, (NCHUNK + nw - 1) // nw, body, 0)


def _sc_gather(x, src2):
    mesh = plsc.VectorSubcoreMesh(core_axis_name="c", subcore_axis_name="s")
    nw = mesh.num_cores * mesh.num_subcores
    return pl.kernel(
        functools.partial(_sc_gather_body, nw),
        out_type=jax.ShapeDtypeStruct((E, AD), _f32),
        mesh=mesh,
        compiler_params=pltpu.CompilerParams(use_tc_tiling_on_sc=False),
        scratch_types=[
            pltpu.VMEM((CHUNK,), jnp.int32),
            pltpu.VMEM((CHUNK, AD), _f32),
            pltpu.SemaphoreType.DMA,
        ],
    )(x, src2)


def _sc_scatter_body(nw, msg_hbm, dst_hbm, zeros_hbm, out_hbm, idx_v, rows_v,
                     acc, sem):
    c = lax.axis_index("c")
    s = lax.axis_index("s")
    nc = nw // SUBS
    wid = s * nc + c

    row0 = s * ROWS_PER_SUB
    pltpu.sync_copy(zeros_hbm.at[pl.ds(row0, ROWS_PER_SUB)],
                    acc.at[pl.ds(row0, ROWS_PER_SUB)])
    plsc.subcore_barrier()

    def body(j, carry):
        cc = j * nw + wid

        @pl.when(cc < NCHUNK)
        def _():
            pltpu.sync_copy(dst_hbm.at[cc], idx_v)
            pltpu.async_copy(msg_hbm.at[pl.ds(cc * CHUNK, CHUNK)], rows_v,
                             sem).wait()
            pltpu.sync_copy(rows_v, acc.at[idx_v], add=True)

        return carry

    lax.fori_loop(0, (NCHUNK + nw - 1) // nw, body, 0)
    plsc.subcore_barrier()
    pltpu.sync_copy(acc.at[pl.ds(row0, ROWS_PER_SUB)],
                    out_hbm.at[c].at[pl.ds(row0, ROWS_PER_SUB)])


def _sc_scatter(msg, dst2, zeros_n):
    mesh = plsc.VectorSubcoreMesh(core_axis_name="c", subcore_axis_name="s")
    nw = mesh.num_cores * mesh.num_subcores
    return pl.kernel(
        functools.partial(_sc_scatter_body, nw),
        out_type=jax.ShapeDtypeStruct((mesh.num_cores, N, AD), _f32),
        mesh=mesh,
        compiler_params=pltpu.CompilerParams(use_tc_tiling_on_sc=False),
        scratch_types=[
            pltpu.VMEM((CHUNK,), jnp.int32),
            pltpu.VMEM((CHUNK, AD), _f32),
            pltpu.VMEM_SHARED((N, AD), _f32),
            pltpu.SemaphoreType.DMA,
        ],
    )(msg, dst2, zeros_n)


def _msg_body(ea_ref, xg_ref, bm1_ref, bm1b_ref, bb_ref, b2_ref, o_ref):
    he = jnp.maximum(_d(ea_ref[...], bm1_ref[...]) + bm1b_ref[...], 0.0)
    w = lax.dot_general(he.astype(_bf16), bb_ref[...],
                        (((1,), (0,)), ((), ())),
                        preferred_element_type=_f32) + b2_ref[...]
    wb = w.astype(_bf16)
    xgb = xg_ref[...].astype(_bf16)
    acc = jnp.zeros((ET, AD), _f32)
    for i in range(AD):
        acc = acc + (xgb[:, i:i + 1].astype(_f32)
                     * wb[:, i * AD:(i + 1) * AD].astype(_f32))
    o_ref[...] = acc


def _msg(ea, xg, bm1_W, bm1_b2, bb, bm2_b2):
    grid = (E // ET,)
    return pl.pallas_call(
        _msg_body,
        grid=grid,
        in_specs=[
            pl.BlockSpec((ET, ED), lambda i: (i, 0)),
            pl.BlockSpec((ET, AD), lambda i: (i, 0)),
            pl.BlockSpec((ED, CD), lambda i: (0, 0)),
            pl.BlockSpec((1, CD), lambda i: (0, 0)),
            pl.BlockSpec((CD, AD * AD), lambda i: (0, 0)),
            pl.BlockSpec((1, AD * AD), lambda i: (0, 0)),
        ],
        out_specs=pl.BlockSpec((ET, AD), lambda i: (i, 0)),
        out_shape=jax.ShapeDtypeStruct((E, AD), _f32),
    )(ea, xg, bm1_W, bm1_b2, bb, bm2_b2)


def _gru_body(macc_ref, cb_ref, h_ref, wih_ref, whh_ref, bih_ref, bhh_ref,
              o_ref):
    m = jnp.maximum(macc_ref[0] + macc_ref[1] + cb_ref[...], 0.0)
    h = h_ref[...]
    gi = _d(m, wih_ref[...]) + bih_ref[...]
    gh = _d(h, whh_ref[...]) + bhh_ref[...]
    r = jax.nn.sigmoid(gi[:, :AD] + gh[:, :AD])
    z = jax.nn.sigmoid(gi[:, AD:2 * AD] + gh[:, AD:2 * AD])
    n = jnp.tanh(gi[:, 2 * AD:] + r * gh[:, 2 * AD:])
    o_ref[...] = (1.0 - z) * n + z * h


def _gru(macc, conv_b2, h, wihT, whhT, bih2, bhh2):
    return pl.pallas_call(
        _gru_body,
        out_shape=jax.ShapeDtypeStruct((N, AD), _f32),
    )(macc, conv_b2, h, wihT, whhT, bih2, bhh2)


def _s2s_body(x_ref, b2_ref, wih_ref, whh_ref, bih_ref, bhh_ref,
              bn_g_ref, bn_b_ref, fc1_W_ref, fc1_b_ref, bn1_g_ref, bn1_b_ref,
              fc2_W_ref, fc2_b_ref, bn2_g_ref, bn2_b_ref, pred_W_ref,
              pred_b_ref, o_ref):
    x = x_ref[...]
    bidx = b2_ref[...]
    mask = bidx == lax.broadcasted_iota(jnp.int32, (1, B), 1)
    mask_bf = mask.astype(_bf16)

    def split3(t):
        th = t.astype(_bf16)
        r1 = t - th.astype(_f32)
        tl = r1.astype(_bf16)
        tq = (r1 - tl.astype(_f32)).astype(_bf16)
        return jnp.concatenate([th, tl, tq], axis=1)

    q_star = jnp.zeros((B, 2 * AD), _f32)
    hl = jnp.zeros((B, AD), _f32)
    cl = jnp.zeros((B, AD), _f32)
    for _ in range(STEPS):
        g = (_d(q_star, wih_ref[...]) + bih_ref[...]
             + _d(hl, whh_ref[...]) + bhh_ref[...])
        cl = (jax.nn.sigmoid(g[:, AD:2 * AD]) * cl
              + jax.nn.sigmoid(g[:, :AD]) * jnp.tanh(g[:, 2 * AD:3 * AD]))
        hl = jax.nn.sigmoid(g[:, 3 * AD:]) * jnp.tanh(cl)
        q = hl
        xq = lax.dot_general(x, q, (((1,), (1,)), ((), ())),
                             precision=_HI)
        e = jnp.sum(jnp.where(mask, xq, 0.0), axis=1, keepdims=True)
        emax = jnp.max(jnp.where(mask, e, -1e30), axis=0, keepdims=True)
        emax = jnp.where(emax <= -1e29, 0.0, emax)
        ex = jnp.exp(e - jnp.sum(jnp.where(mask, emax, 0.0),
                                 axis=1, keepdims=True))
        denom = jnp.sum(jnp.where(mask, ex, 0.0), axis=0, keepdims=True)
        a = ex / (jnp.sum(jnp.where(mask, denom, 0.0),
                          axis=1, keepdims=True) + 1e-16)
        rc = lax.dot_general(mask_bf, split3(a * x), (((0,), (0,)), ((), ())),
                             preferred_element_type=_f32)
        r_ = rc[:, :AD] + rc[:, AD:2 * AD] + rc[:, 2 * AD:]
        q_star = jnp.concatenate([q, r_], axis=1)

    def bn(t, g_, b_):
        mu = jnp.mean(t, axis=0, keepdims=True)
        var = jnp.mean((t - mu) * (t - mu), axis=0, keepdims=True)
        return (t - mu) / jnp.sqrt(var + 1e-5) * g_ + b_

    out = bn(q_star, bn_g_ref[...], bn_b_ref[...])
    out = bn(jnp.maximum(_d(out, fc1_W_ref[...]) + fc1_b_ref[...], 0.0),
             bn1_g_ref[...], bn1_b_ref[...])
    out = bn(jnp.maximum(_d(out, fc2_W_ref[...]) + fc2_b_ref[...], 0.0),
             bn2_g_ref[...], bn2_b_ref[...])
    o_ref[...] = _d(out, pred_W_ref[...]) + pred_b_ref[...]


def _s2s(x, batch2, *weights):
    return pl.pallas_call(
        _s2s_body,
        out_shape=jax.ShapeDtypeStruct((B, OUT), _f32),
    )(x, batch2, *weights)


@jax.jit
def kernel(x, edge_attr, edge_index, batch, lin_W, lin_b, bm1_W, bm1_b,
           bm2_W, bm2_b, conv_b, gru_Wih, gru_Whh, gru_bih, gru_bhh,
           lstm_Wih, lstm_Whh, lstm_bih, lstm_bhh, bn_g, bn_b, fc1_W, fc1_b,
           bn1_g, bn1_b, fc2_W, fc2_b, bn2_g, bn2_b, pred_W, pred_b):
    src2 = edge_index[0].reshape(NCHUNK, CHUNK)
    dst2 = edge_index[1].reshape(NCHUNK, CHUNK)
    batch2 = batch.reshape(N, 1)
    zeros_n = jnp.zeros((N, AD), _f32)

    bb = bm2_W.astype(_bf16)
    bm2_b2 = bm2_b.reshape(1, AD * AD)
    bm1_b2 = bm1_b.reshape(1, CD)
    wihT = gru_Wih.T
    whhT = gru_Whh.T
    bih2 = gru_bih.reshape(1, 3 * AD)
    bhh2 = gru_bhh.reshape(1, 3 * AD)

    x0 = _lin_relu(x, lin_W, lin_b.reshape(1, AD))
    h = x0
    for _ in range(NUM_EMBEDS):
        xg = _sc_gather(h, src2)
        msg = _msg(edge_attr, xg, bm1_W, bm1_b2, bb, bm2_b2)
        macc = _sc_scatter(msg, dst2, zeros_n)
        h = _gru(macc, conv_b.reshape(1, AD), h, wihT, whhT, bih2, bhh2)

    return _s2s(h, batch2,
                lstm_Wih.T, lstm_Whh.T,
                lstm_bih.reshape(1, 4 * AD), lstm_bhh.reshape(1, 4 * AD),
                bn_g.reshape(1, 2 * AD), bn_b.reshape(1, 2 * AD),
                fc1_W, fc1_b.reshape(1, 64),
                bn1_g.reshape(1, 64), bn1_b.reshape(1, 64),
                fc2_W, fc2_b.reshape(1, 64),
                bn2_g.reshape(1, 64), bn2_b.reshape(1, 64),
                pred_W, pred_b.reshape(1, OUT))
